# Initial kernel scaffold; baseline (speedup 1.0000x reference)
import functools

import jax
import jax.numpy as jnp
from jax import lax
from jax.experimental import pallas as pl
from jax.experimental.pallas import tpu as pltpu
from jax.experimental.pallas import tpu_sc as plsc

B = 16384
IN_C = 67
OUT_C = 73
NW = 32
RPW = B // NW
L = 16


def _sc_body(elev_hbm, tab_hbm, out_hbm, in_v, out_v, tab_v):
    wid = lax.axis_index("s") * 2 + lax.axis_index("c")
    base = wid * RPW

    pltpu.sync_copy(elev_hbm.at[pl.ds(base, RPW)], in_v)
    pltpu.sync_copy(tab_hbm, tab_v)

    pltpu.sync_copy(in_v.at[:, pl.ds(0, 16)], out_v.at[:, pl.ds(0, 16)])
    pltpu.sync_copy(in_v.at[:, pl.ds(19, 48)], out_v.at[:, pl.ds(17, 48)])

    lanes = lax.iota(jnp.int32, L)

    def chunk(i, carry):
        rvec = i * L + lanes
        d17 = plsc.load_gather(in_v, [rvec, jnp.full((L,), 17, jnp.int32)])
        plsc.store_scatter(out_v, [rvec, jnp.full((L,), 16, jnp.int32)], d17)
        d18 = plsc.load_gather(in_v, [rvec, jnp.full((L,), 18, jnp.int32)])
        idx8 = d18.astype(jnp.int32) * 8
        for e in range(8):
            vals = plsc.load_gather(tab_v, [idx8 + e])
            plsc.store_scatter(out_v, [rvec, jnp.full((L,), 65 + e, jnp.int32)], vals)
        return carry

    lax.fori_loop(0, RPW // L, chunk, 0)

    pltpu.sync_copy(out_v, out_hbm.at[pl.ds(base, RPW)])


_sc_kernel = functools.partial(
    pl.kernel,
    out_type=jax.ShapeDtypeStruct((B, OUT_C), jnp.float32),
    mesh=plsc.VectorSubcoreMesh(core_axis_name="c", subcore_axis_name="s"),
    scratch_types=[
        pltpu.VMEM((RPW, IN_C), jnp.float32),
        pltpu.VMEM((RPW, OUT_C), jnp.float32),
        pltpu.VMEM((32,), jnp.float32),
    ],
)(_sc_body)


@jax.jit
def kernel(elev_info, door_table, srv_dir_table):
    del srv_dir_table
    return _sc_kernel(elev_info, door_table.reshape(-1))


# speedup vs baseline: 1.0677x; 1.0677x over previous
import functools

import jax
import jax.numpy as jnp
from jax import lax
from jax.experimental import pallas as pl
from jax.experimental.pallas import tpu as pltpu
from jax.experimental.pallas import tpu_sc as plsc

B = 16384
IN_C = 67
OUT_C = 73
NW = 32
RPW = B // NW
L = 16


def _sc_body(elev_hbm, tab_hbm, out_hbm, in_v, out_v, tab_v):
    wid = lax.axis_index("s") * 2 + lax.axis_index("c")

    pltpu.sync_copy(elev_hbm.at[pl.ds(wid * RPW * IN_C, RPW * IN_C)], in_v)
    pltpu.sync_copy(tab_hbm, tab_v)

    def row(r, carry):
        s = r * IN_C
        d = r * OUT_C
        out_v[pl.ds(d, 16)] = in_v[pl.ds(s, 16)]
        out_v[pl.ds(d + 17, 16)] = in_v[pl.ds(s + 19, 16)]
        out_v[pl.ds(d + 33, 16)] = in_v[pl.ds(s + 35, 16)]
        out_v[pl.ds(d + 49, 16)] = in_v[pl.ds(s + 51, 16)]
        return carry

    lax.fori_loop(0, RPW, row, 0)

    lanes = lax.iota(jnp.int32, L)

    def chunk(i, carry):
        rvec = i * L + lanes
        svec = rvec * IN_C
        dvec = rvec * OUT_C
        d17 = plsc.load_gather(in_v, [svec + 17])
        plsc.store_scatter(out_v, [dvec + 16], d17)
        d18 = plsc.load_gather(in_v, [svec + 18])
        idx8 = d18.astype(jnp.int32) * 8
        for e in range(8):
            vals = plsc.load_gather(tab_v, [idx8 + e])
            plsc.store_scatter(out_v, [dvec + 65 + e], vals)
        return carry

    lax.fori_loop(0, RPW // L, chunk, 0)

    pltpu.sync_copy(out_v, out_hbm.at[pl.ds(wid * RPW * OUT_C, RPW * OUT_C)])


_sc_kernel = functools.partial(
    pl.kernel,
    out_type=jax.ShapeDtypeStruct((B * OUT_C,), jnp.float32),
    mesh=plsc.VectorSubcoreMesh(core_axis_name="c", subcore_axis_name="s"),
    compiler_params=pltpu.CompilerParams(needs_layout_passes=False),
    scratch_types=[
        pltpu.VMEM((RPW * IN_C,), jnp.float32),
        pltpu.VMEM((RPW * OUT_C,), jnp.float32),
        pltpu.VMEM((32,), jnp.float32),
    ],
)(_sc_body)


@jax.jit
def kernel(elev_info, door_table, srv_dir_table):
    del srv_dir_table
    out = _sc_kernel(elev_info.reshape(-1), door_table.reshape(-1))
    return out.reshape(B, OUT_C)


# speedup vs baseline: 1.2000x; 1.1238x over previous
import functools

import jax
import jax.numpy as jnp
from jax import lax
from jax.experimental import pallas as pl
from jax.experimental.pallas import tpu as pltpu
from jax.experimental.pallas import tpu_sc as plsc

B = 16384
IN_C = 67
OUT_C = 73
NW = 32
RPW = B // NW
L = 16


def _sc_body(elev_hbm, tab_hbm, out_hbm, in_v, out_v, tab_v, sem_in):
    wid = lax.axis_index("s") * 2 + lax.axis_index("c")
    rows = pl.ds(wid * RPW, RPW)

    cp_in = pltpu.make_async_copy(elev_hbm.at[rows], in_v, sem_in)
    cp_in.start()
    pltpu.sync_copy(tab_hbm, tab_v)
    cp_in.wait()

    @plsc.parallel_loop(0, RPW, unroll=8)
    def row(r):
        out_v[r, pl.ds(0, 16)] = in_v[r, pl.ds(0, 16)]
        out_v[r, pl.ds(17, 16)] = in_v[r, pl.ds(19, 16)]
        out_v[r, pl.ds(33, 16)] = in_v[r, pl.ds(35, 16)]
        out_v[r, pl.ds(49, 16)] = in_v[r, pl.ds(51, 16)]

    lanes = lax.iota(jnp.int32, L)

    @plsc.parallel_loop(0, RPW // L, unroll=4)
    def chunk(i):
        rvec = i * L + lanes
        d17 = plsc.load_gather(in_v, [rvec, jnp.full((L,), 17, jnp.int32)])
        plsc.store_scatter(out_v, [rvec, jnp.full((L,), 16, jnp.int32)], d17)
        d18 = plsc.load_gather(in_v, [rvec, jnp.full((L,), 18, jnp.int32)])
        idx8 = d18.astype(jnp.int32) * 8
        for e in range(8):
            vals = plsc.load_gather(tab_v, [idx8 + e])
            plsc.store_scatter(out_v, [rvec, jnp.full((L,), 65 + e, jnp.int32)], vals)

    pltpu.sync_copy(out_v, out_hbm.at[rows])


_sc_kernel = functools.partial(
    pl.kernel,
    out_type=jax.ShapeDtypeStruct((B, OUT_C), jnp.float32),
    mesh=plsc.VectorSubcoreMesh(core_axis_name="c", subcore_axis_name="s"),
    compiler_params=pltpu.CompilerParams(
        needs_layout_passes=False, use_tc_tiling_on_sc=False),
    scratch_types=[
        pltpu.VMEM((RPW, IN_C), jnp.float32),
        pltpu.VMEM((RPW, OUT_C), jnp.float32),
        pltpu.VMEM((32,), jnp.float32),
        pltpu.SemaphoreType.DMA,
    ],
)(_sc_body)


@jax.jit
def kernel(elev_info, door_table, srv_dir_table):
    del srv_dir_table
    return _sc_kernel(elev_info, door_table.reshape(-1))


# speedup vs baseline: 1.6818x; 1.4015x over previous
import functools

import jax
import jax.numpy as jnp
from jax import lax
from jax.experimental import pallas as pl
from jax.experimental.pallas import tpu as pltpu
from jax.experimental.pallas import tpu_sc as plsc

B = 16384
IN_C = 67
OUT_C = 73
NW = 32
RPW = B // NW
CH = 256
L = 16


def _sc_body(elev_hbm, tab_hbm, out_hbm, in_v, out_v, tab_v, sem_in):
    wid = lax.axis_index("s") * 2 + lax.axis_index("c")
    pltpu.sync_copy(tab_hbm, tab_v)
    lanes = lax.iota(jnp.int32, L)

    for k in range(RPW // CH):
        rows = pl.ds(wid * RPW + k * CH, CH)
        pltpu.sync_copy(elev_hbm.at[rows], in_v)

        @plsc.parallel_loop(0, CH, unroll=8)
        def row(r):
            out_v[r, pl.ds(0, 16)] = in_v[r, pl.ds(0, 16)]
            out_v[r, pl.ds(17, 16)] = in_v[r, pl.ds(19, 16)]
            out_v[r, pl.ds(33, 16)] = in_v[r, pl.ds(35, 16)]
            out_v[r, pl.ds(49, 16)] = in_v[r, pl.ds(51, 16)]

        @plsc.parallel_loop(0, CH // L, unroll=4)
        def chunk(i):
            rvec = i * L + lanes
            d17 = plsc.load_gather(in_v, [rvec, jnp.full((L,), 17, jnp.int32)])
            plsc.store_scatter(out_v, [rvec, jnp.full((L,), 16, jnp.int32)], d17)
            d18 = plsc.load_gather(in_v, [rvec, jnp.full((L,), 18, jnp.int32)])
            idx8 = d18.astype(jnp.int32) * 8
            for e in range(8):
                vals = plsc.load_gather(tab_v, [idx8 + e])
                plsc.store_scatter(
                    out_v, [rvec, jnp.full((L,), 65 + e, jnp.int32)], vals)

        pltpu.sync_copy(out_v, out_hbm.at[rows])


_sc_kernel = functools.partial(
    pl.kernel,
    out_type=jax.ShapeDtypeStruct((B, OUT_C), jnp.float32),
    mesh=plsc.VectorSubcoreMesh(core_axis_name="c", subcore_axis_name="s"),
    compiler_params=pltpu.CompilerParams(
        needs_layout_passes=False, use_tc_tiling_on_sc=True),
    scratch_types=[
        pltpu.VMEM((CH, IN_C), jnp.float32),
        pltpu.VMEM((CH, OUT_C), jnp.float32),
        pltpu.VMEM((32,), jnp.float32),
        pltpu.SemaphoreType.DMA,
    ],
)(_sc_body)


@jax.jit
def kernel(elev_info, door_table, srv_dir_table):
    del srv_dir_table
    return _sc_kernel(elev_info, door_table.reshape(-1))


# speedup vs baseline: 2.7772x; 1.6513x over previous
import functools

import jax
import jax.numpy as jnp
from jax import lax
from jax.experimental import pallas as pl
from jax.experimental.pallas import tpu as pltpu
from jax.experimental.pallas import tpu_sc as plsc

B = 16384
IN_C = 67
OUT_C = 73
NW = 32
CPW = B // NW
L = 16


def _sc_body(elev_t_hbm, tab_hbm, out_t_hbm, in_v, buf, tab_v):
    wid = lax.axis_index("s") * 2 + lax.axis_index("c")
    cols = pl.ds(wid * CPW, CPW)

    pltpu.sync_copy(elev_t_hbm.at[:, cols], in_v)
    pltpu.sync_copy(tab_hbm, tab_v)

    @plsc.parallel_loop(0, CPW // L, unroll=2)
    def chunk(j):
        sl = pl.ds(j * L, L)
        idx8 = in_v[18, sl].astype(jnp.int32) * 8
        for c in range(16):
            buf[c, sl] = in_v[c, sl]
        buf[16, sl] = in_v[17, sl]
        for c in range(17, 65):
            buf[c, sl] = in_v[c + 2, sl]
        for e in range(8):
            buf[65 + e, sl] = plsc.load_gather(tab_v, [idx8 + e])

    pltpu.sync_copy(buf, out_t_hbm.at[:, cols])


_sc_kernel = functools.partial(
    pl.kernel,
    out_type=jax.ShapeDtypeStruct((OUT_C, B), jnp.float32),
    mesh=plsc.VectorSubcoreMesh(core_axis_name="c", subcore_axis_name="s"),
    compiler_params=pltpu.CompilerParams(
        needs_layout_passes=False, use_tc_tiling_on_sc=True),
    scratch_types=[
        pltpu.VMEM((IN_C, CPW), jnp.float32),
        pltpu.VMEM((OUT_C, CPW), jnp.float32),
        pltpu.VMEM((32,), jnp.float32),
    ],
)(_sc_body)


@jax.jit
def kernel(elev_info, door_table, srv_dir_table):
    del srv_dir_table
    out_t = _sc_kernel(elev_info.T, door_table.reshape(-1))
    return out_t.T
